# Initial kernel scaffold; baseline (speedup 1.0000x reference)
import jax
import jax.numpy as jnp
from jax.experimental import pallas as pl


def kernel(loc_data, conf_data, landm_data, priors, targets):
    raise NotImplementedError("write your pallas kernel here")


# speedup vs baseline: 82.4066x; 82.4066x over previous
import jax
import jax.numpy as jnp
from jax.experimental import pallas as pl
from jax.experimental.pallas import tpu as pltpu

_NUM_CLASSES = 2
_THRESH = 0.35
_NEGPOS = 7.0
_V0, _V1 = 0.1, 0.2


def _stage1_body(nobj, p_real, rows, tgt_ref, loc_ref, conf_ref, landm_ref,
                 pri_ref, ce_ref, posf_ref, stats_ref):
    shp = (rows, 128)
    pidx = (jax.lax.broadcasted_iota(jnp.int32, shp, 0) * 128
            + jax.lax.broadcasted_iota(jnp.int32, shp, 1))
    valid = pidx < p_real

    pcx = pri_ref[0]
    pcy = pri_ref[1]
    pw = pri_ref[2]
    ph = pri_ref[3]
    loc = loc_ref[0]
    conf = conf_ref[0]
    landm = landm_ref[0]
    px1 = pcx - pw * 0.5
    py1 = pcy - ph * 0.5
    px2 = pcx + pw * 0.5
    py2 = pcy + ph * 0.5
    area_b = pw * ph

    best = jnp.full(shp, -1.0, jnp.float32)
    bidx = jnp.zeros(shp, jnp.int32)
    bpis = []
    for o in range(nobj):
        tx1 = tgt_ref[0, 0, o * 15 + 0]
        ty1 = tgt_ref[0, 0, o * 15 + 1]
        tx2 = tgt_ref[0, 0, o * 15 + 2]
        ty2 = tgt_ref[0, 0, o * 15 + 3]
        iw = jnp.maximum(jnp.minimum(tx2, px2) - jnp.maximum(tx1, px1), 0.0)
        ih = jnp.maximum(jnp.minimum(ty2, py2) - jnp.maximum(ty1, py1), 0.0)
        inter = iw * ih
        area_a = (tx2 - tx1) * (ty2 - ty1)
        ov = inter / (area_a + area_b - inter)
        ov = jnp.where(valid, ov, -1.0)
        upd = ov > best
        bidx = jnp.where(upd, o, bidx)
        best = jnp.where(upd, ov, best)
        rowmax = jnp.max(ov)
        bpis.append(jnp.min(jnp.where(ov == rowmax, pidx, p_real * 2)))
    for o in range(nobj):
        hit = pidx == bpis[o]
        best = jnp.where(hit, 2.0, best)
        bidx = jnp.where(hit, o, bidx)

    pos = best >= _THRESH
    posf = pos.astype(jnp.float32)

    def gather_truth(col):
        acc = jnp.zeros(shp, jnp.float32)
        for o in range(nobj):
            acc = jnp.where(bidx == o, tgt_ref[0, 0, o * 15 + col], acc)
        return acc

    mx1 = gather_truth(0)
    my1 = gather_truth(1)
    mx2 = gather_truth(2)
    my2 = gather_truth(3)

    gcx = ((mx1 + mx2) * 0.5 - pcx) / (_V0 * pw)
    gcy = ((my1 + my2) * 0.5 - pcy) / (_V0 * ph)
    gw = jnp.log(jnp.maximum((mx2 - mx1) / pw, 1e-30)) * (1.0 / _V1)
    gh = jnp.log(jnp.maximum((my2 - my1) / ph, 1e-30)) * (1.0 / _V1)

    sl = jnp.zeros(shp, jnp.float32)
    for j, g in enumerate((gcx, gcy, gw, gh)):
        d = jnp.abs(loc[j] - g)
        sl += jnp.where(d < 1.0, 0.5 * d * d, d - 0.5)
    loss_l = jnp.sum(sl * posf)

    slm = jnp.zeros(shp, jnp.float32)
    for j in range(10):
        lm = gather_truth(4 + j)
        pc = pcx if j % 2 == 0 else pcy
        pwh = pw if j % 2 == 0 else ph
        g = (lm - pc) / (_V0 * pwh)
        d = jnp.abs(landm[j] - g)
        slm += jnp.where(d < 1.0, 0.5 * d * d, d - 0.5)
    loss_lm = jnp.sum(slm * posf)

    c0 = conf[0]
    c1 = conf[1]
    mx = jnp.maximum(c0, c1)
    lse = mx + jnp.log(jnp.exp(c0 - mx) + jnp.exp(c1 - mx))
    ce = lse - jnp.where(pos, c1, c0)
    ce = jnp.where(valid, ce, -1.0)
    num_pos = jnp.sum(posf)

    ce_ref[0] = ce
    posf_ref[0] = posf
    ri = jax.lax.broadcasted_iota(jnp.int32, (8, 128), 0)
    li = jax.lax.broadcasted_iota(jnp.int32, (8, 128), 1)
    stats = (jnp.where((ri == 0) & (li == 0), loss_l, 0.0)
             + jnp.where((ri == 0) & (li == 1), loss_lm, 0.0)
             + jnp.where((ri == 0) & (li == 2), num_pos, 0.0))
    stats_ref[0] = stats


def _stage2_body(b, p_real, ppad, ce_ref, posf_ref, stats_ref, out_ref):
    ce = ce_ref[...].reshape(b, ppad)
    posf = posf_ref[...].reshape(b, ppad)
    stats = stats_ref[...]

    bits = jax.lax.bitcast_convert_type(ce, jnp.int32)
    key = jnp.where(bits >= 0, bits, bits ^ 0x7FFFFFFF)
    pid = jax.lax.broadcasted_iota(jnp.int32, (b, ppad), 1)

    np_rows = jax.lax.slice(stats, (0, 0, 2), (b, 1, 3)).reshape(b, 1)
    k = jnp.minimum(np_rows * _NEGPOS, float(p_real - 1))

    prefix = jnp.zeros((b, 1), jnp.int32)
    for bit in range(30, -1, -1):
        test = prefix | (1 << bit)
        cnt = jnp.sum((key >= test).astype(jnp.float32), axis=1, keepdims=True)
        prefix = jnp.where(cnt >= k, test, prefix)
    t = jax.lax.bitcast_convert_type(prefix, jnp.float32)

    gt = key > prefix
    cnt_gt = jnp.sum(gt.astype(jnp.float32), axis=1, keepdims=True)
    sum_gt = jnp.sum(jnp.where(gt, ce, 0.0), axis=1, keepdims=True)
    m = k - cnt_gt

    eq = key == prefix
    smax = jnp.zeros((b, 1), jnp.int32)
    for bit in range(14, -1, -1):
        cand = smax | (1 << bit)
        cnt2 = jnp.sum((eq & (pid < cand)).astype(jnp.float32),
                       axis=1, keepdims=True)
        smax = jnp.where(cnt2 < m, cand, smax)

    lt_f = (key < prefix).astype(jnp.float32)
    tie_out = (eq & (pid > smax)).astype(jnp.float32)
    pos_extra = jnp.sum(posf * ce * (lt_f + tie_out), axis=1, keepdims=True)

    loss_c = jnp.sum(sum_gt + m * t + pos_extra)
    loss_l = jnp.sum(jax.lax.slice(stats, (0, 0, 0), (b, 1, 1)))
    loss_lm = jnp.sum(jax.lax.slice(stats, (0, 0, 1), (b, 1, 2)))
    n_tot = jnp.maximum(jnp.sum(np_rows), 1.0)

    ri = jax.lax.broadcasted_iota(jnp.int32, (8, 128), 0)
    li = jax.lax.broadcasted_iota(jnp.int32, (8, 128), 1)
    out = (jnp.where((ri == 0) & (li == 0), loss_l / n_tot, 0.0)
           + jnp.where((ri == 0) & (li == 1), loss_c / n_tot, 0.0)
           + jnp.where((ri == 0) & (li == 2), loss_lm / n_tot, 0.0))
    out_ref[...] = out


def kernel(loc_data, conf_data, landm_data, priors, targets):
    b, p_real, _ = loc_data.shape
    nobj = targets.shape[1]
    rows = (p_real + 127) // 128
    ppad = rows * 128
    pad = ppad - p_real

    def prep(x, k):
        x = jnp.moveaxis(x, 2, 1)
        x = jnp.pad(x, ((0, 0), (0, 0), (0, pad)))
        return x.reshape(b, k, rows, 128)

    loc_t = prep(loc_data, 4)
    conf_t = prep(conf_data, _NUM_CLASSES)
    landm_t = prep(landm_data, 10)
    pri_pad = jnp.broadcast_to(
        jnp.array([[0.5], [0.5], [1.0], [1.0]], jnp.float32), (4, pad))
    pri_t = jnp.concatenate([priors.T, pri_pad], axis=1).reshape(4, rows, 128)
    tgt_flat = targets.reshape(b, 1, nobj * 15)

    import functools
    s1 = functools.partial(_stage1_body, nobj, p_real, rows)
    ce, posf, stats = pl.pallas_call(
        s1,
        grid=(b,),
        in_specs=[
            pl.BlockSpec((1, 1, nobj * 15), lambda i: (i, 0, 0),
                         memory_space=pltpu.SMEM),
            pl.BlockSpec((1, 4, rows, 128), lambda i: (i, 0, 0, 0)),
            pl.BlockSpec((1, _NUM_CLASSES, rows, 128), lambda i: (i, 0, 0, 0)),
            pl.BlockSpec((1, 10, rows, 128), lambda i: (i, 0, 0, 0)),
            pl.BlockSpec((4, rows, 128), lambda i: (0, 0, 0)),
        ],
        out_specs=[
            pl.BlockSpec((1, rows, 128), lambda i: (i, 0, 0)),
            pl.BlockSpec((1, rows, 128), lambda i: (i, 0, 0)),
            pl.BlockSpec((1, 8, 128), lambda i: (i, 0, 0)),
        ],
        out_shape=[
            jax.ShapeDtypeStruct((b, rows, 128), jnp.float32),
            jax.ShapeDtypeStruct((b, rows, 128), jnp.float32),
            jax.ShapeDtypeStruct((b, 8, 128), jnp.float32),
        ],
    )(tgt_flat, loc_t, conf_t, landm_t, pri_t)

    s2 = functools.partial(_stage2_body, b, p_real, ppad)
    out = pl.pallas_call(
        s2,
        out_shape=jax.ShapeDtypeStruct((8, 128), jnp.float32),
    )(ce, posf, stats)

    return (out[0, 0], out[0, 1], out[0, 2])
